# baseline (speedup 1.0000x reference)
import functools

import jax
import jax.numpy as jnp
import numpy as np
from jax.experimental import pallas as pl

_TL = 2048


def _to_T(a):
    L, d = a.shape
    Lp = ((L + _TL - 1) // _TL) * _TL
    return jnp.pad(a.T, ((0, 0), (0, Lp - L)))


def _from_T(aT, L):
    return aT[:, :L].T


def _k_mm_stats(x_ref, w_ref, b_ref, z_ref, s_ref, ss_ref, *, Ltrue):
    i = pl.program_id(0)
    x = x_ref[...]
    z = jax.lax.dot_general(w_ref[...], x, (((0,), (0,)), ((), ())),
                            preferred_element_type=jnp.float32) + b_ref[...]
    z_ref[...] = z
    lane = jax.lax.broadcasted_iota(jnp.int32, (1, x.shape[1]), 1) + i * x.shape[1]
    m = (lane < Ltrue).astype(jnp.float32)
    zm = z * m
    ps = jnp.sum(zm, axis=1, keepdims=True)
    pss = jnp.sum(zm * zm, axis=1, keepdims=True)

    @pl.when(i == 0)
    def _():
        s_ref[...] = ps
        ss_ref[...] = pss

    @pl.when(i != 0)
    def _():
        s_ref[...] = s_ref[...] + ps
        ss_ref[...] = ss_ref[...] + pss


def _k_bn_mm_stats(x_ref, g_ref, be_ref, mn_ref, vr_ref, w_ref, b_ref,
                   z_ref, s_ref, ss_ref, *, Ltrue):
    i = pl.program_id(0)
    x = x_ref[...]
    xb = g_ref[...] * (x - mn_ref[...]) * jax.lax.rsqrt(vr_ref[...] + 1e-5) + be_ref[...]
    xb = jnp.maximum(xb, 0.0)
    z = jax.lax.dot_general(w_ref[...], xb, (((0,), (0,)), ((), ())),
                            preferred_element_type=jnp.float32) + b_ref[...]
    z_ref[...] = z
    lane = jax.lax.broadcasted_iota(jnp.int32, (1, x.shape[1]), 1) + i * x.shape[1]
    m = (lane < Ltrue).astype(jnp.float32)
    zm = z * m
    ps = jnp.sum(zm, axis=1, keepdims=True)
    pss = jnp.sum(zm * zm, axis=1, keepdims=True)

    @pl.when(i == 0)
    def _():
        s_ref[...] = ps
        ss_ref[...] = pss

    @pl.when(i != 0)
    def _():
        s_ref[...] = s_ref[...] + ps
        ss_ref[...] = ss_ref[...] + pss


def _k_bn_apply(x_ref, g_ref, be_ref, mn_ref, vr_ref, o_ref):
    x = x_ref[...]
    xb = g_ref[...] * (x - mn_ref[...]) * jax.lax.rsqrt(vr_ref[...] + 1e-5) + be_ref[...]
    o_ref[...] = jnp.maximum(xb, 0.0)


def _mm_stats(xT, W, b, Ltrue):
    din, Lp = xT.shape
    dout = W.shape[1]
    grid = (Lp // _TL,)
    z, s, ss = pl.pallas_call(
        functools.partial(_k_mm_stats, Ltrue=Ltrue),
        grid=grid,
        in_specs=[
            pl.BlockSpec((din, _TL), lambda i: (0, i)),
            pl.BlockSpec((din, dout), lambda i: (0, 0)),
            pl.BlockSpec((dout, 1), lambda i: (0, 0)),
        ],
        out_specs=[
            pl.BlockSpec((dout, _TL), lambda i: (0, i)),
            pl.BlockSpec((dout, 1), lambda i: (0, 0)),
            pl.BlockSpec((dout, 1), lambda i: (0, 0)),
        ],
        out_shape=[
            jax.ShapeDtypeStruct((dout, Lp), jnp.float32),
            jax.ShapeDtypeStruct((dout, 1), jnp.float32),
            jax.ShapeDtypeStruct((dout, 1), jnp.float32),
        ],
    )(xT, W, b.reshape(dout, 1))
    mn = s / np.float32(Ltrue)
    vr = ss / np.float32(Ltrue) - mn * mn
    return z, mn, vr


def _bn_mm_stats(xT, g, be, mn, vr, W, b, Ltrue):
    din, Lp = xT.shape
    dout = W.shape[1]
    grid = (Lp // _TL,)
    z, s, ss = pl.pallas_call(
        functools.partial(_k_bn_mm_stats, Ltrue=Ltrue),
        grid=grid,
        in_specs=[
            pl.BlockSpec((din, _TL), lambda i: (0, i)),
            pl.BlockSpec((din, 1), lambda i: (0, 0)),
            pl.BlockSpec((din, 1), lambda i: (0, 0)),
            pl.BlockSpec((din, 1), lambda i: (0, 0)),
            pl.BlockSpec((din, 1), lambda i: (0, 0)),
            pl.BlockSpec((din, dout), lambda i: (0, 0)),
            pl.BlockSpec((dout, 1), lambda i: (0, 0)),
        ],
        out_specs=[
            pl.BlockSpec((dout, _TL), lambda i: (0, i)),
            pl.BlockSpec((dout, 1), lambda i: (0, 0)),
            pl.BlockSpec((dout, 1), lambda i: (0, 0)),
        ],
        out_shape=[
            jax.ShapeDtypeStruct((dout, Lp), jnp.float32),
            jax.ShapeDtypeStruct((dout, 1), jnp.float32),
            jax.ShapeDtypeStruct((dout, 1), jnp.float32),
        ],
    )(xT, g.reshape(din, 1), be.reshape(din, 1), mn, vr, W, b.reshape(dout, 1))
    mn2 = s / np.float32(Ltrue)
    vr2 = ss / np.float32(Ltrue) - mn2 * mn2
    return z, mn2, vr2


def _bn_apply(xT, g, be, mn, vr):
    din, Lp = xT.shape
    grid = (Lp // _TL,)
    return pl.pallas_call(
        _k_bn_apply,
        grid=grid,
        in_specs=[
            pl.BlockSpec((din, _TL), lambda i: (0, i)),
            pl.BlockSpec((din, 1), lambda i: (0, 0)),
            pl.BlockSpec((din, 1), lambda i: (0, 0)),
            pl.BlockSpec((din, 1), lambda i: (0, 0)),
            pl.BlockSpec((din, 1), lambda i: (0, 0)),
        ],
        out_specs=pl.BlockSpec((din, _TL), lambda i: (0, i)),
        out_shape=jax.ShapeDtypeStruct((din, Lp), jnp.float32),
    )(xT, g.reshape(din, 1), be.reshape(din, 1), mn, vr)


def _mlp_T(ps, xT, Ltrue):
    (W1, b1, g1, be1), (W2, b2, g2, be2) = ps
    z1, m1, v1 = _mm_stats(xT, W1, b1, Ltrue)
    z2, m2, v2 = _bn_mm_stats(z1, g1, be1, m1, v1, W2, b2, Ltrue)
    return _bn_apply(z2, g2, be2, m2, v2)


def _k_head(x_ref, bt_ref, wm_ref, bm_ref, wl_ref, bl_ref, o_ref):
    x = x_ref[...]
    bt = bt_ref[...]
    cols = []
    for g in range(8):
        mk = (bt == g).astype(jnp.float32)
        sg = jnp.sum(x * mk, axis=1, keepdims=True)
        cg = jnp.maximum(jnp.sum(mk), 1.0)
        cols.append(sg / cg)
    pooled = jnp.concatenate(cols, axis=1)
    h = jax.lax.dot_general(wm_ref[...], pooled, (((0,), (0,)), ((), ())),
                            preferred_element_type=jnp.float32) + bm_ref[...]
    h = jnp.maximum(h, 0.0)
    lo = jax.lax.dot_general(wl_ref[...], h, (((0,), (0,)), ((), ())),
                             preferred_element_type=jnp.float32) + bl_ref[...]
    e = jnp.exp(lo - jnp.max(lo, axis=0, keepdims=True))
    o_ref[...] = e / jnp.sum(e, axis=0, keepdims=True)


def _head(outT, batch_c, M, Wm, bm, Wl, bl):
    din, Lp = outT.shape
    bt = jnp.pad(batch_c, (0, Lp - M), constant_values=-1).reshape(1, Lp)
    o = pl.pallas_call(
        _k_head,
        in_specs=[
            pl.BlockSpec((din, Lp), lambda: (0, 0)),
            pl.BlockSpec((1, Lp), lambda: (0, 0)),
            pl.BlockSpec(Wm.shape, lambda: (0, 0)),
            pl.BlockSpec((64, 1), lambda: (0, 0)),
            pl.BlockSpec(Wl.shape, lambda: (0, 0)),
            pl.BlockSpec((2, 1), lambda: (0, 0)),
        ],
        out_specs=pl.BlockSpec((2, 8), lambda: (0, 0)),
        out_shape=jax.ShapeDtypeStruct((2, 8), jnp.float32),
    )(outT, bt, Wm, bm.reshape(64, 1), Wl, bl.reshape(2, 1))
    return o.T


def _lieconv(pos, feats, batch, ratio, K, lp, gp, wp, is_first):
    N = pos.shape[0]
    M = int(ratio * N)
    fps_idx = (jnp.arange(M, dtype=jnp.int32) * N) // M
    pos_c = pos[fps_idx]
    batch_c = batch[fps_idx]
    d2 = jnp.sum((pos_c[:, None, :] - pos[None, :, :]) ** 2, axis=-1)
    d2 = jnp.where(batch_c[:, None] == batch[None, :], d2, jnp.float32(1e10))
    _, nn_idx = jax.lax.top_k(-d2, K)
    pos_j = pos[nn_idx]
    rel = pos_j - pos_c[:, None, :]
    r = jnp.sqrt(jnp.sum(rel ** 2, axis=-1, keepdims=True) + 1e-12)
    coords = jnp.concatenate([rel, r], axis=-1)
    fj = pos_j if is_first else feats[nn_idx]
    inp = jnp.concatenate([coords, fj], axis=-1)
    MK = M * K
    hT = _mlp_T(lp, _to_T(inp.reshape(MK, inp.shape[-1])), MK)
    wT = _mlp_T(wp, _to_T(coords.reshape(MK, 3)), MK)
    h = _from_T(hT, MK).reshape(M, K, -1)
    w = _from_T(wT, MK).reshape(M, K, -1)
    agg = jnp.einsum('mkc,mkd->mcd', h, w) / np.float32(K)
    outT = _mlp_T(gp, _to_T(agg.reshape(M, -1)), M)
    out = _from_T(outT, M)
    return out, pos_c, batch_c


def kernel(pos, x, batch, params):
    out, pos, batch = _lieconv(pos, None, batch, 0.75, 32,
                               params['local1'], params['global1'],
                               params['weight1'], True)
    out, pos, batch = _lieconv(pos, out, batch, 0.75, 32,
                               params['local2'], params['global2'],
                               params['weight2'], False)
    out, pos, batch = _lieconv(pos, out, batch, 0.6, 64,
                               params['local3'], params['global3'],
                               params['weight3'], False)
    M = out.shape[0]
    Wm, bm = params['mlp_lin']
    Wl, bl = params['out_lin']
    return _head(_to_T(out), batch, M, Wm, bm, Wl, bl)


# speedup vs baseline: 2.2847x; 2.2847x over previous
import functools

import jax
import jax.numpy as jnp
import numpy as np
from jax.experimental import pallas as pl

_TL = 2048


def _to_T(a):
    L, d = a.shape
    Lp = ((L + _TL - 1) // _TL) * _TL
    return jnp.pad(a.T, ((0, 0), (0, Lp - L)))


def _from_T(aT, L):
    return aT[:, :L].T


def _k_mm_stats(x_ref, w_ref, b_ref, z_ref, s_ref, ss_ref, *, Ltrue):
    i = pl.program_id(0)
    x = x_ref[...]
    z = jax.lax.dot_general(w_ref[...], x, (((0,), (0,)), ((), ())),
                            preferred_element_type=jnp.float32) + b_ref[...]
    z_ref[...] = z
    lane = jax.lax.broadcasted_iota(jnp.int32, (1, x.shape[1]), 1) + i * x.shape[1]
    m = (lane < Ltrue).astype(jnp.float32)
    zm = z * m
    ps = jnp.sum(zm, axis=1, keepdims=True)
    pss = jnp.sum(zm * zm, axis=1, keepdims=True)

    @pl.when(i == 0)
    def _():
        s_ref[...] = ps
        ss_ref[...] = pss

    @pl.when(i != 0)
    def _():
        s_ref[...] = s_ref[...] + ps
        ss_ref[...] = ss_ref[...] + pss


def _k_bn_mm_stats(x_ref, g_ref, be_ref, mn_ref, vr_ref, w_ref, b_ref,
                   z_ref, s_ref, ss_ref, *, Ltrue):
    i = pl.program_id(0)
    x = x_ref[...]
    xb = g_ref[...] * (x - mn_ref[...]) * jax.lax.rsqrt(vr_ref[...] + 1e-5) + be_ref[...]
    xb = jnp.maximum(xb, 0.0)
    z = jax.lax.dot_general(w_ref[...], xb, (((0,), (0,)), ((), ())),
                            preferred_element_type=jnp.float32) + b_ref[...]
    z_ref[...] = z
    lane = jax.lax.broadcasted_iota(jnp.int32, (1, x.shape[1]), 1) + i * x.shape[1]
    m = (lane < Ltrue).astype(jnp.float32)
    zm = z * m
    ps = jnp.sum(zm, axis=1, keepdims=True)
    pss = jnp.sum(zm * zm, axis=1, keepdims=True)

    @pl.when(i == 0)
    def _():
        s_ref[...] = ps
        ss_ref[...] = pss

    @pl.when(i != 0)
    def _():
        s_ref[...] = s_ref[...] + ps
        ss_ref[...] = ss_ref[...] + pss


def _k_bn_apply(x_ref, g_ref, be_ref, mn_ref, vr_ref, o_ref):
    x = x_ref[...]
    xb = g_ref[...] * (x - mn_ref[...]) * jax.lax.rsqrt(vr_ref[...] + 1e-5) + be_ref[...]
    o_ref[...] = jnp.maximum(xb, 0.0)


def _mm_stats(xT, W, b, Ltrue):
    din, Lp = xT.shape
    dout = W.shape[1]
    grid = (Lp // _TL,)
    z, s, ss = pl.pallas_call(
        functools.partial(_k_mm_stats, Ltrue=Ltrue),
        grid=grid,
        in_specs=[
            pl.BlockSpec((din, _TL), lambda i: (0, i)),
            pl.BlockSpec((din, dout), lambda i: (0, 0)),
            pl.BlockSpec((dout, 1), lambda i: (0, 0)),
        ],
        out_specs=[
            pl.BlockSpec((dout, _TL), lambda i: (0, i)),
            pl.BlockSpec((dout, 1), lambda i: (0, 0)),
            pl.BlockSpec((dout, 1), lambda i: (0, 0)),
        ],
        out_shape=[
            jax.ShapeDtypeStruct((dout, Lp), jnp.float32),
            jax.ShapeDtypeStruct((dout, 1), jnp.float32),
            jax.ShapeDtypeStruct((dout, 1), jnp.float32),
        ],
    )(xT, W, b.reshape(dout, 1))
    mn = s / np.float32(Ltrue)
    vr = ss / np.float32(Ltrue) - mn * mn
    return z, mn, vr


def _bn_mm_stats(xT, g, be, mn, vr, W, b, Ltrue):
    din, Lp = xT.shape
    dout = W.shape[1]
    grid = (Lp // _TL,)
    z, s, ss = pl.pallas_call(
        functools.partial(_k_bn_mm_stats, Ltrue=Ltrue),
        grid=grid,
        in_specs=[
            pl.BlockSpec((din, _TL), lambda i: (0, i)),
            pl.BlockSpec((din, 1), lambda i: (0, 0)),
            pl.BlockSpec((din, 1), lambda i: (0, 0)),
            pl.BlockSpec((din, 1), lambda i: (0, 0)),
            pl.BlockSpec((din, 1), lambda i: (0, 0)),
            pl.BlockSpec((din, dout), lambda i: (0, 0)),
            pl.BlockSpec((dout, 1), lambda i: (0, 0)),
        ],
        out_specs=[
            pl.BlockSpec((dout, _TL), lambda i: (0, i)),
            pl.BlockSpec((dout, 1), lambda i: (0, 0)),
            pl.BlockSpec((dout, 1), lambda i: (0, 0)),
        ],
        out_shape=[
            jax.ShapeDtypeStruct((dout, Lp), jnp.float32),
            jax.ShapeDtypeStruct((dout, 1), jnp.float32),
            jax.ShapeDtypeStruct((dout, 1), jnp.float32),
        ],
    )(xT, g.reshape(din, 1), be.reshape(din, 1), mn, vr, W, b.reshape(dout, 1))
    mn2 = s / np.float32(Ltrue)
    vr2 = ss / np.float32(Ltrue) - mn2 * mn2
    return z, mn2, vr2


def _bn_apply(xT, g, be, mn, vr):
    din, Lp = xT.shape
    grid = (Lp // _TL,)
    return pl.pallas_call(
        _k_bn_apply,
        grid=grid,
        in_specs=[
            pl.BlockSpec((din, _TL), lambda i: (0, i)),
            pl.BlockSpec((din, 1), lambda i: (0, 0)),
            pl.BlockSpec((din, 1), lambda i: (0, 0)),
            pl.BlockSpec((din, 1), lambda i: (0, 0)),
            pl.BlockSpec((din, 1), lambda i: (0, 0)),
        ],
        out_specs=pl.BlockSpec((din, _TL), lambda i: (0, i)),
        out_shape=jax.ShapeDtypeStruct((din, Lp), jnp.float32),
    )(xT, g.reshape(din, 1), be.reshape(din, 1), mn, vr)


def _mlp_T(ps, xT, Ltrue):
    (W1, b1, g1, be1), (W2, b2, g2, be2) = ps
    z1, m1, v1 = _mm_stats(xT, W1, b1, Ltrue)
    z2, m2, v2 = _bn_mm_stats(z1, g1, be1, m1, v1, W2, b2, Ltrue)
    return _bn_apply(z2, g2, be2, m2, v2)


_TMK = 128


def _k_knn(pc_ref, bc_ref, pos_ref, bt_ref, nn_ref, *, K):
    pcx = pc_ref[:, 0:1]
    pcy = pc_ref[:, 1:2]
    px = pos_ref[0:1, :]
    py = pos_ref[1:2, :]
    dx = pcx - px
    dy = pcy - py
    d2 = dx * dx + dy * dy
    same = bc_ref[...] == bt_ref[...]
    d2 = jnp.where(same, d2, jnp.float32(1e10))
    lane = jax.lax.broadcasted_iota(jnp.int32, d2.shape, 1)
    cols = []
    for _ in range(K):
        mn = jnp.min(d2, axis=1, keepdims=True)
        idx = jnp.min(jnp.where(d2 <= mn, lane, jnp.int32(2 ** 30)),
                      axis=1, keepdims=True)
        cols.append(idx)
        d2 = jnp.where(lane == idx, jnp.float32(3e38), d2)
    nn_ref[...] = jnp.concatenate(cols, axis=1)


def _knn(pos_c, batch_c, pos, batch, K):
    M = pos_c.shape[0]
    N = pos.shape[0]
    Np = ((N + 127) // 128) * 128
    posT = jnp.pad(pos.T, ((0, 0), (0, Np - N)))
    btT = jnp.pad(batch, (0, Np - N), constant_values=-1).reshape(1, Np)
    grid = ((M + _TMK - 1) // _TMK,)
    nn = pl.pallas_call(
        functools.partial(_k_knn, K=K),
        grid=grid,
        in_specs=[
            pl.BlockSpec((_TMK, 2), lambda i: (i, 0)),
            pl.BlockSpec((_TMK, 1), lambda i: (i, 0)),
            pl.BlockSpec((2, Np), lambda i: (0, 0)),
            pl.BlockSpec((1, Np), lambda i: (0, 0)),
        ],
        out_specs=pl.BlockSpec((_TMK, K), lambda i: (i, 0)),
        out_shape=jax.ShapeDtypeStruct((M, K), jnp.int32),
    )(pos_c, batch_c.reshape(M, 1), posT, btT)
    return nn


def _k_head(x_ref, bt_ref, wm_ref, bm_ref, wl_ref, bl_ref, o_ref):
    x = x_ref[...]
    bt = bt_ref[...]
    cols = []
    for g in range(8):
        mk = (bt == g).astype(jnp.float32)
        sg = jnp.sum(x * mk, axis=1, keepdims=True)
        cg = jnp.maximum(jnp.sum(mk), 1.0)
        cols.append(sg / cg)
    pooled = jnp.concatenate(cols, axis=1)
    h = jax.lax.dot_general(wm_ref[...], pooled, (((0,), (0,)), ((), ())),
                            preferred_element_type=jnp.float32) + bm_ref[...]
    h = jnp.maximum(h, 0.0)
    lo = jax.lax.dot_general(wl_ref[...], h, (((0,), (0,)), ((), ())),
                             preferred_element_type=jnp.float32) + bl_ref[...]
    e = jnp.exp(lo - jnp.max(lo, axis=0, keepdims=True))
    o_ref[...] = e / jnp.sum(e, axis=0, keepdims=True)


def _head(outT, batch_c, M, Wm, bm, Wl, bl):
    din, Lp = outT.shape
    bt = jnp.pad(batch_c, (0, Lp - M), constant_values=-1).reshape(1, Lp)
    o = pl.pallas_call(
        _k_head,
        in_specs=[
            pl.BlockSpec((din, Lp), lambda: (0, 0)),
            pl.BlockSpec((1, Lp), lambda: (0, 0)),
            pl.BlockSpec(Wm.shape, lambda: (0, 0)),
            pl.BlockSpec((64, 1), lambda: (0, 0)),
            pl.BlockSpec(Wl.shape, lambda: (0, 0)),
            pl.BlockSpec((2, 1), lambda: (0, 0)),
        ],
        out_specs=pl.BlockSpec((2, 8), lambda: (0, 0)),
        out_shape=jax.ShapeDtypeStruct((2, 8), jnp.float32),
    )(outT, bt, Wm, bm.reshape(64, 1), Wl, bl.reshape(2, 1))
    return o.T


def _lieconv(pos, feats, batch, ratio, K, lp, gp, wp, is_first):
    N = pos.shape[0]
    M = int(ratio * N)
    fps_idx = (jnp.arange(M, dtype=jnp.int32) * N) // M
    pos_c = pos[fps_idx]
    batch_c = batch[fps_idx]
    nn_idx = _knn(pos_c, batch_c, pos, batch, K)
    pos_j = pos[nn_idx]
    rel = pos_j - pos_c[:, None, :]
    r = jnp.sqrt(jnp.sum(rel ** 2, axis=-1, keepdims=True) + 1e-12)
    coords = jnp.concatenate([rel, r], axis=-1)
    fj = pos_j if is_first else feats[nn_idx]
    inp = jnp.concatenate([coords, fj], axis=-1)
    MK = M * K
    hT = _mlp_T(lp, _to_T(inp.reshape(MK, inp.shape[-1])), MK)
    wT = _mlp_T(wp, _to_T(coords.reshape(MK, 3)), MK)
    h = _from_T(hT, MK).reshape(M, K, -1)
    w = _from_T(wT, MK).reshape(M, K, -1)
    agg = jnp.einsum('mkc,mkd->mcd', h, w) / np.float32(K)
    outT = _mlp_T(gp, _to_T(agg.reshape(M, -1)), M)
    out = _from_T(outT, M)
    return out, pos_c, batch_c


def kernel(pos, x, batch, params):
    out, pos, batch = _lieconv(pos, None, batch, 0.75, 32,
                               params['local1'], params['global1'],
                               params['weight1'], True)
    out, pos, batch = _lieconv(pos, out, batch, 0.75, 32,
                               params['local2'], params['global2'],
                               params['weight2'], False)
    out, pos, batch = _lieconv(pos, out, batch, 0.6, 64,
                               params['local3'], params['global3'],
                               params['weight3'], False)
    M = out.shape[0]
    Wm, bm = params['mlp_lin']
    Wl, bl = params['out_lin']
    return _head(_to_T(out), batch, M, Wm, bm, Wl, bl)


# speedup vs baseline: 2.3714x; 1.0379x over previous
import functools

import jax
import jax.numpy as jnp
import numpy as np
from jax.experimental import pallas as pl

_TL = 2048


def _to_T(a):
    L, d = a.shape
    Lp = ((L + _TL - 1) // _TL) * _TL
    return jnp.pad(a.T, ((0, 0), (0, Lp - L)))


def _from_T(aT, L):
    return aT[:, :L].T


def _k_mm_stats(x_ref, w_ref, b_ref, z_ref, s_ref, ss_ref, *, Ltrue):
    i = pl.program_id(0)
    x = x_ref[...]
    z = jax.lax.dot_general(w_ref[...], x, (((0,), (0,)), ((), ())),
                            preferred_element_type=jnp.float32) + b_ref[...]
    z_ref[...] = z
    lane = jax.lax.broadcasted_iota(jnp.int32, (1, x.shape[1]), 1) + i * x.shape[1]
    m = (lane < Ltrue).astype(jnp.float32)
    zm = z * m
    ps = jnp.sum(zm, axis=1, keepdims=True)
    pss = jnp.sum(zm * zm, axis=1, keepdims=True)

    @pl.when(i == 0)
    def _():
        s_ref[...] = ps
        ss_ref[...] = pss

    @pl.when(i != 0)
    def _():
        s_ref[...] = s_ref[...] + ps
        ss_ref[...] = ss_ref[...] + pss


def _k_bn_mm_stats(x_ref, g_ref, be_ref, mn_ref, vr_ref, w_ref, b_ref,
                   z_ref, s_ref, ss_ref, *, Ltrue):
    i = pl.program_id(0)
    x = x_ref[...]
    xb = g_ref[...] * (x - mn_ref[...]) * jax.lax.rsqrt(vr_ref[...] + 1e-5) + be_ref[...]
    xb = jnp.maximum(xb, 0.0)
    z = jax.lax.dot_general(w_ref[...], xb, (((0,), (0,)), ((), ())),
                            preferred_element_type=jnp.float32) + b_ref[...]
    z_ref[...] = z
    lane = jax.lax.broadcasted_iota(jnp.int32, (1, x.shape[1]), 1) + i * x.shape[1]
    m = (lane < Ltrue).astype(jnp.float32)
    zm = z * m
    ps = jnp.sum(zm, axis=1, keepdims=True)
    pss = jnp.sum(zm * zm, axis=1, keepdims=True)

    @pl.when(i == 0)
    def _():
        s_ref[...] = ps
        ss_ref[...] = pss

    @pl.when(i != 0)
    def _():
        s_ref[...] = s_ref[...] + ps
        ss_ref[...] = ss_ref[...] + pss


def _k_bn_apply(x_ref, g_ref, be_ref, mn_ref, vr_ref, o_ref):
    x = x_ref[...]
    xb = g_ref[...] * (x - mn_ref[...]) * jax.lax.rsqrt(vr_ref[...] + 1e-5) + be_ref[...]
    o_ref[...] = jnp.maximum(xb, 0.0)


def _mm_stats(xT, W, b, Ltrue):
    din, Lp = xT.shape
    dout = W.shape[1]
    grid = (Lp // _TL,)
    z, s, ss = pl.pallas_call(
        functools.partial(_k_mm_stats, Ltrue=Ltrue),
        grid=grid,
        in_specs=[
            pl.BlockSpec((din, _TL), lambda i: (0, i)),
            pl.BlockSpec((din, dout), lambda i: (0, 0)),
            pl.BlockSpec((dout, 1), lambda i: (0, 0)),
        ],
        out_specs=[
            pl.BlockSpec((dout, _TL), lambda i: (0, i)),
            pl.BlockSpec((dout, 1), lambda i: (0, 0)),
            pl.BlockSpec((dout, 1), lambda i: (0, 0)),
        ],
        out_shape=[
            jax.ShapeDtypeStruct((dout, Lp), jnp.float32),
            jax.ShapeDtypeStruct((dout, 1), jnp.float32),
            jax.ShapeDtypeStruct((dout, 1), jnp.float32),
        ],
    )(xT, W, b.reshape(dout, 1))
    mn = s / np.float32(Ltrue)
    vr = ss / np.float32(Ltrue) - mn * mn
    return z, mn, vr


def _bn_mm_stats(xT, g, be, mn, vr, W, b, Ltrue):
    din, Lp = xT.shape
    dout = W.shape[1]
    grid = (Lp // _TL,)
    z, s, ss = pl.pallas_call(
        functools.partial(_k_bn_mm_stats, Ltrue=Ltrue),
        grid=grid,
        in_specs=[
            pl.BlockSpec((din, _TL), lambda i: (0, i)),
            pl.BlockSpec((din, 1), lambda i: (0, 0)),
            pl.BlockSpec((din, 1), lambda i: (0, 0)),
            pl.BlockSpec((din, 1), lambda i: (0, 0)),
            pl.BlockSpec((din, 1), lambda i: (0, 0)),
            pl.BlockSpec((din, dout), lambda i: (0, 0)),
            pl.BlockSpec((dout, 1), lambda i: (0, 0)),
        ],
        out_specs=[
            pl.BlockSpec((dout, _TL), lambda i: (0, i)),
            pl.BlockSpec((dout, 1), lambda i: (0, 0)),
            pl.BlockSpec((dout, 1), lambda i: (0, 0)),
        ],
        out_shape=[
            jax.ShapeDtypeStruct((dout, Lp), jnp.float32),
            jax.ShapeDtypeStruct((dout, 1), jnp.float32),
            jax.ShapeDtypeStruct((dout, 1), jnp.float32),
        ],
    )(xT, g.reshape(din, 1), be.reshape(din, 1), mn, vr, W, b.reshape(dout, 1))
    mn2 = s / np.float32(Ltrue)
    vr2 = ss / np.float32(Ltrue) - mn2 * mn2
    return z, mn2, vr2


def _bn_apply(xT, g, be, mn, vr):
    din, Lp = xT.shape
    grid = (Lp // _TL,)
    return pl.pallas_call(
        _k_bn_apply,
        grid=grid,
        in_specs=[
            pl.BlockSpec((din, _TL), lambda i: (0, i)),
            pl.BlockSpec((din, 1), lambda i: (0, 0)),
            pl.BlockSpec((din, 1), lambda i: (0, 0)),
            pl.BlockSpec((din, 1), lambda i: (0, 0)),
            pl.BlockSpec((din, 1), lambda i: (0, 0)),
        ],
        out_specs=pl.BlockSpec((din, _TL), lambda i: (0, i)),
        out_shape=jax.ShapeDtypeStruct((din, Lp), jnp.float32),
    )(xT, g.reshape(din, 1), be.reshape(din, 1), mn, vr)


def _mlp_T(ps, xT, Ltrue):
    (W1, b1, g1, be1), (W2, b2, g2, be2) = ps
    z1, m1, v1 = _mm_stats(xT, W1, b1, Ltrue)
    z2, m2, v2 = _bn_mm_stats(z1, g1, be1, m1, v1, W2, b2, Ltrue)
    return _bn_apply(z2, g2, be2, m2, v2)


_TMK = 256


def _k_knn(pc_ref, bc_ref, pos_ref, bt_ref, nn_ref, *, K):
    pcx = pc_ref[:, 0:1]
    pcy = pc_ref[:, 1:2]
    px = pos_ref[0:1, :]
    py = pos_ref[1:2, :]
    dx = pcx - px
    dy = pcy - py
    d2 = dx * dx + dy * dy
    same = bc_ref[...] == bt_ref[...]
    d2 = jnp.where(same, d2, jnp.float32(1e10))
    lane = jax.lax.broadcasted_iota(jnp.int32, d2.shape, 1)
    cols = []
    for _ in range(K):
        mn = jnp.min(d2, axis=1, keepdims=True)
        idx = jnp.min(jnp.where(d2 <= mn, lane, jnp.int32(2 ** 30)),
                      axis=1, keepdims=True)
        cols.append(idx)
        d2 = jnp.where(lane == idx, jnp.float32(3e38), d2)
    nn_ref[...] = jnp.concatenate(cols, axis=1)


def _knn(pos_c, batch_c, pos, batch, K):
    M = pos_c.shape[0]
    N = pos.shape[0]
    Np = ((N + 127) // 128) * 128
    posT = jnp.pad(pos.T, ((0, 0), (0, Np - N)))
    btT = jnp.pad(batch, (0, Np - N), constant_values=-1).reshape(1, Np)
    grid = ((M + _TMK - 1) // _TMK,)
    nn = pl.pallas_call(
        functools.partial(_k_knn, K=K),
        grid=grid,
        in_specs=[
            pl.BlockSpec((_TMK, 2), lambda i: (i, 0)),
            pl.BlockSpec((_TMK, 1), lambda i: (i, 0)),
            pl.BlockSpec((2, Np), lambda i: (0, 0)),
            pl.BlockSpec((1, Np), lambda i: (0, 0)),
        ],
        out_specs=pl.BlockSpec((_TMK, K), lambda i: (i, 0)),
        out_shape=jax.ShapeDtypeStruct((M, K), jnp.int32),
    )(pos_c, batch_c.reshape(M, 1), posT, btT)
    return nn


def _k_head(x_ref, bt_ref, wm_ref, bm_ref, wl_ref, bl_ref, o_ref):
    x = x_ref[...]
    bt = bt_ref[...]
    cols = []
    for g in range(8):
        mk = (bt == g).astype(jnp.float32)
        sg = jnp.sum(x * mk, axis=1, keepdims=True)
        cg = jnp.maximum(jnp.sum(mk), 1.0)
        cols.append(sg / cg)
    pooled = jnp.concatenate(cols, axis=1)
    h = jax.lax.dot_general(wm_ref[...], pooled, (((0,), (0,)), ((), ())),
                            preferred_element_type=jnp.float32) + bm_ref[...]
    h = jnp.maximum(h, 0.0)
    lo = jax.lax.dot_general(wl_ref[...], h, (((0,), (0,)), ((), ())),
                             preferred_element_type=jnp.float32) + bl_ref[...]
    e = jnp.exp(lo - jnp.max(lo, axis=0, keepdims=True))
    o_ref[...] = e / jnp.sum(e, axis=0, keepdims=True)


def _head(outT, batch_c, M, Wm, bm, Wl, bl):
    din, Lp = outT.shape
    bt = jnp.pad(batch_c, (0, Lp - M), constant_values=-1).reshape(1, Lp)
    o = pl.pallas_call(
        _k_head,
        in_specs=[
            pl.BlockSpec((din, Lp), lambda: (0, 0)),
            pl.BlockSpec((1, Lp), lambda: (0, 0)),
            pl.BlockSpec(Wm.shape, lambda: (0, 0)),
            pl.BlockSpec((64, 1), lambda: (0, 0)),
            pl.BlockSpec(Wl.shape, lambda: (0, 0)),
            pl.BlockSpec((2, 1), lambda: (0, 0)),
        ],
        out_specs=pl.BlockSpec((2, 8), lambda: (0, 0)),
        out_shape=jax.ShapeDtypeStruct((2, 8), jnp.float32),
    )(outT, bt, Wm, bm.reshape(64, 1), Wl, bl.reshape(2, 1))
    return o.T


def _lieconv(pos, feats, batch, ratio, K, lp, gp, wp, is_first):
    N = pos.shape[0]
    M = int(ratio * N)
    fps_idx = (jnp.arange(M, dtype=jnp.int32) * N) // M
    pos_c = pos[fps_idx]
    batch_c = batch[fps_idx]
    nn_idx = _knn(pos_c, batch_c, pos, batch, K)
    pos_j = pos[nn_idx]
    rel = pos_j - pos_c[:, None, :]
    r = jnp.sqrt(jnp.sum(rel ** 2, axis=-1, keepdims=True) + 1e-12)
    coords = jnp.concatenate([rel, r], axis=-1)
    fj = pos_j if is_first else feats[nn_idx]
    inp = jnp.concatenate([coords, fj], axis=-1)
    MK = M * K
    hT = _mlp_T(lp, _to_T(inp.reshape(MK, inp.shape[-1])), MK)
    wT = _mlp_T(wp, _to_T(coords.reshape(MK, 3)), MK)
    h = _from_T(hT, MK).reshape(M, K, -1)
    w = _from_T(wT, MK).reshape(M, K, -1)
    agg = jnp.einsum('mkc,mkd->mcd', h, w) / np.float32(K)
    outT = _mlp_T(gp, _to_T(agg.reshape(M, -1)), M)
    out = _from_T(outT, M)
    return out, pos_c, batch_c


def kernel(pos, x, batch, params):
    out, pos, batch = _lieconv(pos, None, batch, 0.75, 32,
                               params['local1'], params['global1'],
                               params['weight1'], True)
    out, pos, batch = _lieconv(pos, out, batch, 0.75, 32,
                               params['local2'], params['global2'],
                               params['weight2'], False)
    out, pos, batch = _lieconv(pos, out, batch, 0.6, 64,
                               params['local3'], params['global3'],
                               params['weight3'], False)
    M = out.shape[0]
    Wm, bm = params['mlp_lin']
    Wl, bl = params['out_lin']
    return _head(_to_T(out), batch, M, Wm, bm, Wl, bl)
